# Initial kernel scaffold; baseline (speedup 1.0000x reference)
import jax
import jax.numpy as jnp
from jax.experimental import pallas as pl


def kernel(indices, embeddings):
    raise NotImplementedError("write your pallas kernel here")


# speedup vs baseline: 1.1134x; 1.1134x over previous
import functools

import jax
import jax.numpy as jnp
from jax import lax
from jax.experimental import pallas as pl
from jax.experimental.pallas import tpu as pltpu
from jax.experimental.pallas import tpu_sc as plsc

VOCAB = 1000000
EMBED_DIM = 32
BATCH = 16384
HIST_LEN = 50

TOTAL = BATCH * HIST_LEN
NUM_WORKERS = 32
PER_WORKER = TOTAL // NUM_WORKERS
CHUNK = 1280
NUM_CHUNKS = PER_WORKER // CHUNK


def _gather_call(idx, table):
    mesh = plsc.VectorSubcoreMesh(core_axis_name="c", subcore_axis_name="s")

    @functools.partial(
        pl.kernel,
        mesh=mesh,
        out_type=jax.ShapeDtypeStruct((TOTAL, EMBED_DIM), jnp.float32),
        scratch_types=[
            pltpu.VMEM((PER_WORKER,), jnp.int32),
            pltpu.VMEM((CHUNK, EMBED_DIM), jnp.float32),
            pltpu.VMEM((CHUNK, EMBED_DIM), jnp.float32),
            pltpu.SemaphoreType.DMA,
            pltpu.SemaphoreType.DMA,
        ],
        compiler_params=pltpu.CompilerParams(use_tc_tiling_on_sc=False),
    )
    def k(idx_hbm, table_hbm, out_hbm, idx_v, rows0, rows1, sem0, sem1):
        wid = lax.axis_index("s") * 2 + lax.axis_index("c")
        base = wid * PER_WORKER
        pltpu.sync_copy(idx_hbm.at[pl.ds(base, PER_WORKER)], idx_v)

        rows = (rows0, rows1)
        sems = (sem0, sem1)

        def fire(ci):
            return pltpu.async_copy(
                table_hbm.at[idx_v.at[pl.ds(ci * CHUNK, CHUNK)]],
                rows[ci % 2], sems[ci % 2])

        pending = [fire(0), fire(1)]
        for ci in range(NUM_CHUNKS):
            b = ci % 2
            pending[b].wait()
            pltpu.sync_copy(rows[b], out_hbm.at[pl.ds(base + ci * CHUNK, CHUNK)])
            if ci + 2 < NUM_CHUNKS:
                pending[b] = fire(ci + 2)

    return k(idx, table)


def kernel(indices, embeddings):
    out = _gather_call(indices.reshape(TOTAL), embeddings)
    return out.reshape(BATCH, HIST_LEN, EMBED_DIM)


# speedup vs baseline: 1.5103x; 1.3564x over previous
import functools

import jax
import jax.numpy as jnp
from jax import lax
from jax.experimental import pallas as pl
from jax.experimental.pallas import tpu as pltpu
from jax.experimental.pallas import tpu_sc as plsc

VOCAB = 1000000
EMBED_DIM = 32
BATCH = 16384
HIST_LEN = 50

NUM_WORKERS = 32
B_PER_W = BATCH // NUM_WORKERS
BT_PER_W = B_PER_W // 128


def _gather_call(idx_t, table):
    mesh = plsc.VectorSubcoreMesh(core_axis_name="c", subcore_axis_name="s")

    @functools.partial(
        pl.kernel,
        mesh=mesh,
        out_type=jax.ShapeDtypeStruct(
            (HIST_LEN, EMBED_DIM // 8, BATCH // 128, 8, 128), jnp.float32),
        scratch_types=[
            pltpu.VMEM((HIST_LEN, B_PER_W), jnp.int32),
            pltpu.VMEM((B_PER_W, EMBED_DIM), jnp.float32),
            pltpu.VMEM((B_PER_W, EMBED_DIM), jnp.float32),
            pltpu.VMEM((EMBED_DIM // 8, BT_PER_W, 8, 128), jnp.float32),
            pltpu.VMEM((EMBED_DIM // 8, BT_PER_W, 8, 128), jnp.float32),
            pltpu.SemaphoreType.DMA,
            pltpu.SemaphoreType.DMA,
            pltpu.SemaphoreType.DMA,
            pltpu.SemaphoreType.DMA,
        ],
        compiler_params=pltpu.CompilerParams(
            use_tc_tiling_on_sc=False, needs_layout_passes=False),
    )
    def k(idx_hbm, table_hbm, out_hbm, idx_v, g0, g1, t0, t1,
          gs0, gs1, ss0, ss1):
        wid = lax.axis_index("s") * 2 + lax.axis_index("c")
        b0 = wid * B_PER_W
        pltpu.sync_copy(idx_hbm.at[:, pl.ds(b0, B_PER_W)], idx_v)

        gbuf = (g0, g1)
        tbuf = (t0, t1)
        gsem = (gs0, gs1)
        ssem = (ss0, ss1)
        lane = lax.iota(jnp.int32, 16)

        def fire_gather(h, p):
            pltpu.async_copy(table_hbm.at[idx_v.at[h]], gbuf[p], gsem[p])

        def out_slice(h):
            return out_hbm.at[h, :, pl.ds(wid * BT_PER_W, BT_PER_W), :, :]

        fire_gather(0, 0)
        fire_gather(1, 1)

        def body(i, carry):
            for p in (0, 1):
                h = 2 * i + p
                pltpu.make_async_copy(
                    table_hbm.at[pl.ds(0, B_PER_W)], gbuf[p], gsem[p]).wait()
                @pl.when(i > 0)
                def _():
                    pltpu.make_async_copy(
                        tbuf[p], out_slice(h), ssem[p]).wait()
                g, t = gbuf[p], tbuf[p]
                for eg in range(EMBED_DIM // 8):
                    for es in range(8):
                        colv = jnp.full((16,), eg * 8 + es, jnp.int32)
                        for bt in range(BT_PER_W):
                            for blg in range(8):
                                rowv = (bt * 128 + blg * 16) + lane
                                t[eg, bt, es, pl.ds(blg * 16, 16)] = (
                                    plsc.load_gather(g, [rowv, colv]))
                @pl.when(h + 2 < HIST_LEN)
                def _():
                    fire_gather(h + 2, p)
                pltpu.async_copy(tbuf[p], out_slice(h), ssem[p])
            return carry

        lax.fori_loop(0, HIST_LEN // 2, body, 0)
        for p in (0, 1):
            pltpu.make_async_copy(
                tbuf[p], out_slice(HIST_LEN - 2 + p), ssem[p]).wait()

    return k(idx_t, table)


def kernel(indices, embeddings):
    x = _gather_call(indices.T, embeddings)
    return x.transpose(2, 4, 0, 1, 3).reshape(BATCH, HIST_LEN, EMBED_DIM)
